# Initial kernel scaffold; baseline (speedup 1.0000x reference)
import jax
import jax.numpy as jnp
from jax.experimental import pallas as pl


def kernel(x_user, x_item, edge_index_ui, edge_index_iu, params):
    raise NotImplementedError("write your pallas kernel here")


# speedup vs baseline: 10.3943x; 10.3943x over previous
import functools

import jax
import jax.numpy as jnp
from jax import lax
from jax.experimental import pallas as pl
from jax.experimental.pallas import tpu as pltpu
from jax.experimental.pallas import tpu_sc as plsc

HC = 128
N = 10000
NP = 10240
E_TOTAL = 320000
ROWS = E_TOTAL // 128
RPT = 160
ROWSP = RPT * 16
CH = 80
SCALE = float(HC) ** 0.5

_MESH = dict(core_axis_name="c", subcore_axis_name="s")


def _tile_rows(s):
    base = s * RPT
    cnt = jnp.clip(ROWS - base, 0, RPT)
    return base, cnt


def _counts_body(dui, diu, out, idx_v, ones_v, z_v, cnt_sh):
    c = lax.axis_index("c")
    s = lax.axis_index("s")
    for g in range(8):
        ones_v[pl.ds(g * 16, 16)] = jnp.ones((16,), jnp.float32)
    def zfill(i, _):
        z_v[pl.ds(i * 16, 16)] = jnp.zeros((16,), jnp.float32)
        return 0
    lax.fori_loop(0, 40, zfill, 0)
    pltpu.sync_copy(z_v, cnt_sh.at[pl.ds(s * 640, 640)])
    plsc.subcore_barrier()

    def run(dref):
        base, cnt = _tile_rows(s)
        pltpu.sync_copy(dref.at[pl.ds(base, RPT)], idx_v)
        def row(j, _):
            @pl.when(j < cnt)
            def _():
                pltpu.sync_copy(ones_v, cnt_sh.at[idx_v.at[j]], add=True)
            return 0
        lax.fori_loop(0, RPT, row, 0)

    @pl.when(c == 0)
    def _():
        run(dui)

    @pl.when(c == 1)
    def _():
        run(diu)

    plsc.subcore_barrier()
    pltpu.sync_copy(cnt_sh.at[pl.ds(s * 640, 640)],
                    out.at[pl.ds(c * NP + s * 640, 640)])


def _counts(dui2d, diu2d):
    k = functools.partial(
        pl.kernel,
        out_type=jax.ShapeDtypeStruct((2 * NP,), jnp.float32),
        compiler_params=pltpu.CompilerParams(needs_layout_passes=False),
        mesh=plsc.VectorSubcoreMesh(**_MESH),
        scratch_types=[
            pltpu.VMEM((RPT, 128), jnp.int32),
            pltpu.VMEM((128,), jnp.float32),
            pltpu.VMEM((640,), jnp.float32),
            pltpu.VMEM_SHARED((NP,), jnp.float32),
        ],
    )(_counts_body)
    return k(dui2d, diu2d)


HALF = 5120
ZROWS = HALF + 8


def _attn_body(sui, dui, siu, diu, ru, ri, vu, vi,
               z_ui_o, s_ui_o, z_iu_o, s_iu_o,
               src_v, dst_v, dloc_v, r_v, m_v, msl_v, vrow_v, co_v,
               m_sh, z_sh, s_sh, sem):
    c = lax.axis_index("c")
    s = lax.axis_index("s")
    NEG = jnp.float32(-3.4e38)

    def run(src2d, dst2d, r_hbm, v_hbm, z_out, s_out):
        pltpu.sync_copy(r_hbm, r_v)
        def initm(i, _):
            m_v[pl.ds(i * 16, 16)] = jnp.full((16,), NEG, jnp.float32)
            return 0
        lax.fori_loop(0, NP // 16, initm, 0)
        for g in range(8):
            co_v[pl.ds(g * 16, 16)] = jnp.zeros((16,), jnp.float32)
        for kk in range(5):
            pltpu.sync_copy(co_v, s_sh.at[pl.ds(s * 640 + kk * 128, 128)])
        base, cnt = _tile_rows(s)

        def stage(ch):
            pltpu.sync_copy(src2d.at[pl.ds(base + ch * CH, CH)], src_v)
            pltpu.sync_copy(dst2d.at[pl.ds(base + ch * CH, CH)], dst_v)
            return jnp.clip(cnt - ch * CH, 0, CH)

        for ch in range(2):
            ccnt = stage(ch)
            def maxrow(j, _):
                @pl.when(j < ccnt)
                def _():
                    for g in range(8):
                        s16 = src_v[j, pl.ds(g * 16, 16)]
                        d16 = dst_v[j, pl.ds(g * 16, 16)]
                        rv = plsc.load_gather(r_v, [s16])
                        cur = plsc.load_gather(m_v, [d16])
                        need = rv > cur
                        def wbody(msk):
                            plsc.store_scatter(m_v, [d16], rv, mask=msk)
                            cur2 = plsc.load_gather(m_v, [d16])
                            return jnp.logical_and(msk, rv > cur2)
                        lax.while_loop(lambda msk: jnp.any(msk), wbody, need)
                return 0
            lax.fori_loop(0, CH, maxrow, 0)

        pltpu.sync_copy(m_v, m_sh.at[s])
        plsc.subcore_barrier()
        for kk in range(5):
            pltpu.sync_copy(m_sh.at[:, pl.ds(s * 640 + kk * 128, 128)],
                            vrow_v.at[pl.ds(0, 16)])
            for i in range(8):
                acc = vrow_v[0, pl.ds(i * 16, 16)]
                for t in range(1, 16):
                    acc = jnp.maximum(acc, vrow_v[t, pl.ds(i * 16, 16)])
                msl_v[pl.ds(kk * 128 + i * 16, 16)] = acc
        pltpu.sync_copy(msl_v, m_sh.at[0, pl.ds(s * 640, 640)])
        plsc.subcore_barrier()
        pltpu.sync_copy(m_sh.at[0], m_v)

        for ch in range(2):
            ccnt = stage(ch)
            def coeff(j, _):
                @pl.when(j < ccnt)
                def _():
                    for g in range(8):
                        s16 = src_v[j, pl.ds(g * 16, 16)]
                        d16 = dst_v[j, pl.ds(g * 16, 16)]
                        rv = plsc.load_gather(r_v, [s16])
                        mv = plsc.load_gather(m_v, [d16])
                        cv = jnp.exp(rv - mv)
                        co_v[pl.ds(g * 16, 16)] = cv
                    pltpu.sync_copy(co_v, s_sh.at[dst_v.at[j]], add=True)
                return 0
            lax.fori_loop(0, CH, coeff, 0)

        for h in range(2):
            lo = h * HALF
            def zvr(i, _):
                q, g = i // 8, i % 8
                vrow_v[q, pl.ds(g * 16, 16)] = jnp.zeros((16,), jnp.float32)
                return 0
            lax.fori_loop(0, 128 * 8, zvr, 0)
            for q in range(5):
                pltpu.sync_copy(vrow_v.at[pl.ds(0, 64)],
                                z_sh.at[pl.ds(s * 320 + q * 64, 64)])
            @pl.when(s == 0)
            def _():
                pltpu.sync_copy(vrow_v.at[pl.ds(0, 8)],
                                z_sh.at[pl.ds(HALF, 8)])
            plsc.subcore_barrier()

            for ch in range(2):
                ccnt = stage(ch)
                def att(j, _):
                    @pl.when(j < ccnt)
                    def _():
                        for g in range(8):
                            s16 = src_v[j, pl.ds(g * 16, 16)]
                            d16 = dst_v[j, pl.ds(g * 16, 16)]
                            inr = jnp.logical_and(d16 >= lo, d16 < lo + HALF)
                            dl = jnp.where(inr, d16 - lo, HALF)
                            dloc_v[0, pl.ds(g * 16, 16)] = dl
                            rv = plsc.load_gather(r_v, [s16])
                            mv = plsc.load_gather(m_v, [d16])
                            co_v[pl.ds(g * 16, 16)] = jnp.exp(rv - mv)
                        pltpu.async_copy(v_hbm.at[src_v.at[j]], vrow_v,
                                         sem).wait()
                        def scal(i, _):
                            cb = plsc.load_gather(
                                co_v, [jnp.full((16,), i, jnp.int32)])
                            for g in range(8):
                                vrow_v[i, pl.ds(g * 16, 16)] = (
                                    vrow_v[i, pl.ds(g * 16, 16)] * cb)
                            return 0
                        lax.fori_loop(0, 128, scal, 0)
                        pltpu.sync_copy(vrow_v, z_sh.at[dloc_v.at[0]],
                                        add=True)
                    return 0
                lax.fori_loop(0, CH, att, 0)
            plsc.subcore_barrier()
            pltpu.sync_copy(z_sh.at[pl.ds(s * 320, 320)],
                            z_out.at[pl.ds(lo + s * 320, 320)])
            plsc.subcore_barrier()
        pltpu.sync_copy(s_sh.at[pl.ds(s * 640, 640)],
                        s_out.at[pl.ds(s * 640, 640)])

    @pl.when(c == 0)
    def _():
        run(sui, dui, ru, vu, z_ui_o, s_ui_o)

    @pl.when(c == 1)
    def _():
        run(siu, diu, ri, vi, z_iu_o, s_iu_o)


def _attn(sui2d, dui2d, siu2d, diu2d, ru_p, ri_p, vu, vi):
    k = functools.partial(
        pl.kernel,
        out_type=[
            pltpu.HBM((NP, HC), jnp.float32),
            pltpu.HBM((NP,), jnp.float32),
            pltpu.HBM((NP, HC), jnp.float32),
            pltpu.HBM((NP,), jnp.float32),
        ],
        compiler_params=pltpu.CompilerParams(needs_layout_passes=False),
        mesh=plsc.VectorSubcoreMesh(**_MESH),
        scratch_types=[
            pltpu.VMEM((CH, 128), jnp.int32),
            pltpu.VMEM((CH, 128), jnp.int32),
            pltpu.VMEM((1, 128), jnp.int32),
            pltpu.VMEM((NP,), jnp.float32),
            pltpu.VMEM((NP,), jnp.float32),
            pltpu.VMEM((640,), jnp.float32),
            pltpu.VMEM((128, HC), jnp.float32),
            pltpu.VMEM((128,), jnp.float32),
            pltpu.VMEM_SHARED((16, NP), jnp.float32),
            pltpu.VMEM_SHARED((ZROWS, HC), jnp.float32),
            pltpu.VMEM_SHARED((NP,), jnp.float32),
            pltpu.SemaphoreType.DMA,
        ],
    )(_attn_body)
    return k(sui2d, dui2d, siu2d, diu2d, ru_p, ri_p, vu, vi)


def _dense_tc_body(xu_ref, xi_ref, cu_ref, ci_ref,
                   a_ref, b_ref, qwu_ref, qwi_ref, ks_ref, vu_ref, vi_ref):
    xu = xu_ref[...]
    xi = xi_ref[...]
    b = b_ref[...]
    k1 = jnp.dot(xi, a_ref[0], preferred_element_type=jnp.float32) + b[0:1]
    ks_ref[0:1] = jnp.sum(cu_ref[...] * k1, axis=0, keepdims=True)
    q1 = jnp.dot(xu, a_ref[1], preferred_element_type=jnp.float32) + b[1:2]
    qwu_ref[...] = jnp.dot(q1, a_ref[2], preferred_element_type=jnp.float32)
    vu_ref[...] = jnp.dot(xu, a_ref[3], preferred_element_type=jnp.float32) \
        + b[2:3]
    k2 = jnp.dot(xu, a_ref[4], preferred_element_type=jnp.float32) + b[3:4]
    ks_ref[1:2] = jnp.sum(ci_ref[...] * k2, axis=0, keepdims=True)
    q2 = jnp.dot(xi, a_ref[5], preferred_element_type=jnp.float32) + b[4:5]
    qwi_ref[...] = jnp.dot(q2, a_ref[6], preferred_element_type=jnp.float32)
    vi_ref[...] = jnp.dot(xi, a_ref[7], preferred_element_type=jnp.float32) \
        + b[5:6]


def _dense_tc(xu, xi, cu, ci, a, b):
    return pl.pallas_call(
        _dense_tc_body,
        out_shape=[
            jax.ShapeDtypeStruct((N, HC), jnp.float32),
            jax.ShapeDtypeStruct((N, HC), jnp.float32),
            jax.ShapeDtypeStruct((2, HC), jnp.float32),
            jax.ShapeDtypeStruct((N, HC), jnp.float32),
            jax.ShapeDtypeStruct((N, HC), jnp.float32),
        ],
        compiler_params=pltpu.CompilerParams(
            vmem_limit_bytes=100 * 1024 * 1024),
    )(xu, xi, cu, ci, a, b)


def _norm_tc_body(zu_ref, su_ref, zi_ref, si_ref, ou_ref, oi_ref):
    eps = jnp.float32(1e-16)
    ou_ref[...] = zu_ref[...] / (su_ref[...] + eps)
    oi_ref[...] = zi_ref[...] / (si_ref[...] + eps)


def _norm_tc(zu, su, zi, si):
    return pl.pallas_call(
        _norm_tc_body,
        out_shape=[
            jax.ShapeDtypeStruct((N, HC), jnp.float32),
            jax.ShapeDtypeStruct((N, HC), jnp.float32),
        ],
    )(zu, su, zi, si)


def _stack_params(p):
    a = jnp.stack([
        p["k_item_w"].T, p["q_user_w"].T, p["W_ui"], p["v_user_w"].T,
        p["k_user_w"].T, p["q_item_w"].T, p["W_iu"], p["v_item_w"].T,
    ])
    b = jnp.stack([
        p["k_item_b"], p["q_user_b"], p["v_user_b"],
        p["k_user_b"], p["q_item_b"], p["v_item_b"],
    ])
    return a, b


def _pad_rows(x2d):
    return jnp.pad(x2d, ((0, ROWSP - ROWS), (0, 0)))


def _pad_vec(v):
    return jnp.pad(v, (0, NP - N))


def kernel(x_user, x_item, edge_index_ui, edge_index_iu, params):
    sui2d = _pad_rows(edge_index_ui[0].reshape(ROWS, 128).astype(jnp.int32))
    dui2d = _pad_rows(edge_index_ui[1].reshape(ROWS, 128).astype(jnp.int32))
    siu2d = _pad_rows(edge_index_iu[0].reshape(ROWS, 128).astype(jnp.int32))
    diu2d = _pad_rows(edge_index_iu[1].reshape(ROWS, 128).astype(jnp.int32))

    counts = _counts(dui2d, diu2d).reshape(2, NP)
    cnt_ui = counts[0, :N, None]
    cnt_iu = counts[1, :N, None]

    a1, b1 = _stack_params(params["layer1"])
    a2, b2 = _stack_params(params["layer2"])
    eps = jnp.float32(1e-16)

    qwu, qwi, ks, vu, vi = _dense_tc(x_user, x_item, cnt_ui, cnt_iu, a1, b1)
    ru = jnp.dot(qwu, ks[0]) / SCALE
    ri = jnp.dot(qwi, ks[1]) / SCALE
    z_ui, s_ui, z_iu, s_iu = _attn(sui2d, dui2d, siu2d, diu2d,
                                   _pad_vec(ru), _pad_vec(ri), vu, vi)
    xu2 = z_iu[:N] / (s_iu[:N, None] + eps)
    xi2 = z_ui[:N] / (s_ui[:N, None] + eps)
    qwu2, qwi2, ks2, vu2, vi2 = _dense_tc(xu2, xi2, cnt_ui, cnt_iu, a2, b2)
    ru2 = jnp.dot(qwu2, ks2[0]) / SCALE
    ri2 = jnp.dot(qwi2, ks2[1]) / SCALE
    z_ui2, s_ui2, z_iu2, s_iu2 = _attn(sui2d, dui2d, siu2d, diu2d,
                                       _pad_vec(ru2), _pad_vec(ri2),
                                       vu2, vi2)
    out_user, out_item = _norm_tc(z_iu2[:N], s_iu2[:N, None],
                                  z_ui2[:N], s_ui2[:N, None])
    return (out_user, out_item)
